# Initial kernel scaffold; baseline (speedup 1.0000x reference)
import jax
import jax.numpy as jnp
from jax.experimental import pallas as pl


def kernel(logits, labels):
    raise NotImplementedError("write your pallas kernel here")


# speedup vs baseline: 189.0338x; 189.0338x over previous
import jax
import jax.numpy as jnp
from jax.experimental import pallas as pl
from jax.experimental.pallas import tpu as pltpu

N_BINS = 15
BATCH = 16384
NUM_CLASSES = 1000
PADDED_C = 1024
BN = 1024
GRID = BATCH // BN

_BOUNDARIES = [float(v) for v in jnp.linspace(0.0, 1.0, N_BINS + 1)]


def _ece_kernel(labels_ref, logits_ref, over_ref, under_ref,
                cnt_ref, csum_ref, acc_ref):
    i = pl.program_id(0)
    boundaries = _BOUNDARIES

    x = logits_ref[...]
    m = jnp.max(x, axis=1, keepdims=True)
    e = jnp.exp(x - m)
    s = jnp.sum(e, axis=1, keepdims=True)
    conf = e / s

    cnt_rows = []
    csum_rows = []
    for k in range(N_BINS + 1):
        gt = (conf > boundaries[k]).astype(jnp.float32)
        cnt_rows.append(jnp.sum(gt, axis=0))
        csum_rows.append(jnp.sum(conf * gt, axis=0))
    cnt_part = jnp.stack(cnt_rows, axis=0)
    csum_part = jnp.stack(csum_rows, axis=0)

    lab = labels_ref[i]
    lab_col = lab.reshape(BN, 1)
    cids = jax.lax.broadcasted_iota(jnp.int32, (1, PADDED_C), 1)
    lab_mask = (lab_col == cids).astype(jnp.float32)
    conf_true = jnp.sum(conf * lab_mask, axis=1, keepdims=True)
    jt = jnp.zeros((BN, 1), jnp.int32)
    for k in range(N_BINS + 1):
        jt = jt + (conf_true > boundaries[k]).astype(jnp.int32)
    bins16 = jax.lax.broadcasted_iota(jnp.int32, (1, 16), 1)
    onehot_bin = (jt == bins16 + 1).astype(jnp.float32)
    acc_part = jax.lax.dot_general(
        onehot_bin, lab_mask, (((0,), (0,)), ((), ())),
        preferred_element_type=jnp.float32)

    @pl.when(i == 0)
    def _():
        cnt_ref[...] = cnt_part
        csum_ref[...] = csum_part
        acc_ref[...] = acc_part

    @pl.when(i > 0)
    def _():
        cnt_ref[...] += cnt_part
        csum_ref[...] += csum_part
        acc_ref[...] += acc_part

    @pl.when(i == GRID - 1)
    def _():
        cum_cnt = cnt_ref[...]
        cum_csum = csum_ref[...]
        acc = acc_ref[...]
        zrow = jnp.zeros((1, PADDED_C), jnp.float32)
        count = cum_cnt - jnp.concatenate([cum_cnt[1:], zrow], axis=0)
        conf_sum = cum_csum - jnp.concatenate([cum_csum[1:], zrow], axis=0)
        denom = jnp.maximum(count, 1.0)
        diff = conf_sum / denom - acc / denom
        contrib = jnp.abs(diff) * (count * (1.0 / BATCH))
        num_classes_t = jnp.max(labels_ref[...]) + 1
        active = (cids < num_classes_t).astype(jnp.float32)
        nonempty = count > 0
        over_bc = jnp.where(nonempty & (diff > 0), contrib, 0.0) * active
        under_bc = jnp.where(nonempty & (diff <= 0), contrib, 0.0) * active
        over_ref[...] = jnp.broadcast_to(
            jnp.sum(over_bc, axis=1, keepdims=True), (16, 128))
        under_ref[...] = jnp.broadcast_to(
            jnp.sum(under_bc, axis=1, keepdims=True), (16, 128))


def kernel(logits, labels):
    logits_padded = jnp.pad(
        logits, ((0, 0), (0, PADDED_C - NUM_CLASSES)), constant_values=-1e30)
    labels2d = labels.reshape(GRID, BN)
    over, under = pl.pallas_call(
        _ece_kernel,
        grid=(GRID,),
        in_specs=[
            pl.BlockSpec((GRID, BN), lambda i: (0, 0)),
            pl.BlockSpec((BN, PADDED_C), lambda i: (i, 0)),
        ],
        out_specs=[
            pl.BlockSpec((16, 128), lambda i: (0, 0)),
            pl.BlockSpec((16, 128), lambda i: (0, 0)),
        ],
        out_shape=[
            jax.ShapeDtypeStruct((16, 128), jnp.float32),
            jax.ShapeDtypeStruct((16, 128), jnp.float32),
        ],
        scratch_shapes=[
            pltpu.VMEM((16, PADDED_C), jnp.float32),
            pltpu.VMEM((16, PADDED_C), jnp.float32),
            pltpu.VMEM((16, PADDED_C), jnp.float32),
        ],
    )(labels2d, logits_padded)
    boundaries = jnp.linspace(0.0, 1.0, N_BINS + 1)
    return over[:N_BINS, 0], under[:N_BINS, 0], boundaries[:-1]
